# baseline (speedup 1.0000x reference)
import functools

import jax
import jax.numpy as jnp
from jax import lax
from jax.experimental import pallas as pl
from jax.experimental.pallas import tpu as pltpu
from jax.experimental.pallas import tpu_sc as plsc

B, D, ARITY, OUT = 1024, 512, 8, 16
NMASK = 1 << ARITY
NC, NS, L = 2, 16, 16
NW = NC * NS
D_PER_W = D // NW
GROUPS = B // L

_ALL4 = (NMASK - 1) * OUT
NEG_INF = float("-inf")
_XB = ARITY * B
_PB = NMASK * OUT
_OB = B * OUT


def _sc_body(xt_hbm, pr_hbm, out_hbm, xv, ova, ovb, pv, sin0, sin1,
             sout0, sout1):
    wid = lax.axis_index("s") * NC + lax.axis_index("c")
    d0 = wid * D_PER_W
    sins = (sin0, sin1)
    souts = (sout0, sout1)
    ovs = (ova, ovb)

    def start_in(t, d):
        return (
            pltpu.async_copy(xt_hbm.at[d], xv.at[pl.ds(t * _XB, _XB)], sins[t]),
            pltpu.async_copy(pr_hbm.at[d], pv.at[pl.ds(t * _PB, _PB)], sins[t]),
        )

    descs = start_in(0, d0)
    out_descs = [None, None]
    for j in range(D_PER_W):
        t = j & 1
        if j + 1 < D_PER_W:
            next_descs = start_in(t ^ 1, d0 + j + 1)
        for dsc in descs:
            dsc.wait()
        if out_descs[t] is not None:
            out_descs[t].wait()
        odst = ovs[t]

        def per_group(g, _):
            b0 = g * L
            iota = lax.iota(jnp.int32, L)
            x = [xv[pl.ds(t * _XB + a * B + b0, L)] for a in range(ARITY)]
            mk = [jnp.full((L,), (1 << a) * OUT, jnp.int32)
                  for a in range(ARITY)]
            prev = [jnp.full((L,), NEG_INF) for _ in range(ARITY)]
            zero = jnp.zeros((L,), jnp.int32)
            ninf = jnp.full((L,), NEG_INF)
            for a2 in range(ARITY):
                for a in range(a2 + 1, ARITY):
                    le = x[a2] <= x[a]
                    mk[a2] = mk[a2] | jnp.where(le, (1 << a) * OUT, zero)
                    mk[a] = mk[a] | jnp.where(le, zero, (1 << a2) * OUT)
                    prev[a] = jnp.maximum(prev[a], jnp.where(le, x[a2], ninf))
                    prev[a2] = jnp.maximum(prev[a2], jnp.where(le, ninf, x[a]))
            ck = [x[a] - jnp.where(mk[a] == _ALL4, jnp.float32(0), prev[a])
                  for a in range(ARITY)]
            pref = pv.at[pl.ds(t * _PB, _PB)]
            for s in range(L):
                acc = None
                for k in range(ARITY):
                    idx = iota + jax.lax.broadcast(mk[k][s], (L,))
                    row = plsc.load_gather(pref, [idx])
                    term = jax.lax.broadcast(ck[k][s], (L,)) * row
                    acc = term if acc is None else acc + term
                odst[b0 + s, :] = acc
            return 0

        lax.fori_loop(0, GROUPS, per_group, 0)
        out_descs[t] = pltpu.async_copy(
            odst, out_hbm.at[:, pl.ds((d0 + j) * OUT, OUT)], souts[t])
        if j + 1 < D_PER_W:
            descs = next_descs
    out_descs[0].wait()
    out_descs[1].wait()


@jax.jit
def kernel(X, params):
    xt = jnp.transpose(X, (1, 2, 0)).reshape(D, ARITY * B)
    pr = params.reshape(D, NMASK * OUT)

    run = pl.kernel(
        _sc_body,
        out_type=jax.ShapeDtypeStruct((B, D * OUT), jnp.float32),
        mesh=plsc.VectorSubcoreMesh(core_axis_name="c", subcore_axis_name="s"),
        compiler_params=pltpu.CompilerParams(
            needs_layout_passes=False, use_tc_tiling_on_sc=False),
        scratch_types=[
            pltpu.VMEM((2 * _XB,), jnp.float32),
            pltpu.VMEM((B, OUT), jnp.float32),
            pltpu.VMEM((B, OUT), jnp.float32),
            pltpu.VMEM((2 * _PB,), jnp.float32),
            pltpu.SemaphoreType.DMA,
            pltpu.SemaphoreType.DMA,
            pltpu.SemaphoreType.DMA,
            pltpu.SemaphoreType.DMA,
        ],
    )
    return run(xt, pr).reshape(B, D, OUT)


# speedup vs baseline: 1.4180x; 1.4180x over previous
import functools

import jax
import jax.numpy as jnp
from jax import lax
from jax.experimental import pallas as pl
from jax.experimental.pallas import tpu as pltpu
from jax.experimental.pallas import tpu_sc as plsc

B, D, ARITY, OUT = 1024, 512, 8, 16
NMASK = 1 << ARITY
NC, NS, L = 2, 16, 16
NW = NC * NS
D_PER_W = D // NW
GROUPS = B // L

_ALL4 = (NMASK - 1) * OUT
NEG_INF = float("-inf")
_XB = ARITY * B
_PB = NMASK * OUT
_OB = B * OUT


def _sc_body(xt_hbm, pr_hbm, out_hbm, xv, pv, ov, sin0, sin1, sout0, sout1):
    wid = lax.axis_index("s") * NC + lax.axis_index("c")
    d0 = wid * D_PER_W
    sins = (sin0, sin1)
    souts = (sout0, sout1)

    def in_descs(t, d):
        return (
            pltpu.make_async_copy(
                xt_hbm.at[d], xv.at[pl.ds(t * _XB, _XB)], sins[t]),
            pltpu.make_async_copy(
                pr_hbm.at[d], pv.at[pl.ds(t * _PB, _PB)], sins[t]),
        )

    def out_desc(t, d):
        return pltpu.make_async_copy(
            ov.at[pl.ds(t * _OB, _OB)], out_hbm.at[d], souts[t])

    def network(t, g):
        b0 = g * L
        x = [xv[pl.ds(t * _XB + a * B + b0, L)] for a in range(ARITY)]
        mk = [jnp.full((L,), (1 << a) * OUT, jnp.int32) for a in range(ARITY)]
        prev = [jnp.full((L,), NEG_INF) for _ in range(ARITY)]
        zero = jnp.zeros((L,), jnp.int32)
        ninf = jnp.full((L,), NEG_INF)
        for a2 in range(ARITY):
            for a in range(a2 + 1, ARITY):
                le = x[a2] <= x[a]
                mk[a2] = mk[a2] | jnp.where(le, (1 << a) * OUT, zero)
                mk[a] = mk[a] | jnp.where(le, zero, (1 << a2) * OUT)
                prev[a] = jnp.maximum(prev[a], jnp.where(le, x[a2], ninf))
                prev[a2] = jnp.maximum(prev[a2], jnp.where(le, ninf, x[a]))
        ck = [x[a] - jnp.where(mk[a] == _ALL4, jnp.float32(0), prev[a])
              for a in range(ARITY)]
        return tuple(mk) + tuple(ck)

    def combine(t, g, mkck):
        b0 = g * L
        iota = lax.iota(jnp.int32, L)
        mk, ck = mkck[:ARITY], mkck[ARITY:]
        pref = pv.at[pl.ds(t * _PB, _PB)]
        for s in range(L):
            acc = None
            for k in range(ARITY):
                idx = iota + jax.lax.broadcast(mk[k][s], (L,))
                row = plsc.load_gather(pref, [idx])
                term = jax.lax.broadcast(ck[k][s], (L,)) * row
                acc = term if acc is None else acc + term
            ov[pl.ds(t * _OB + (b0 + s) * OUT, OUT)] = acc

    def per_d(t, j, first):
        d = d0 + j
        dn = jnp.minimum(d + 1, D - 1)
        for dsc in in_descs(t ^ 1, dn):
            dsc.start()
        for dsc in in_descs(t, d):
            dsc.wait()

        @pl.when(jnp.logical_not(first))
        def _():
            out_desc(t, d).wait()

        def body(g, carry):
            nxt = network(t, jnp.minimum(g + 1, GROUPS - 1))
            combine(t, g, carry)
            return nxt

        lax.fori_loop(0, GROUPS, body, network(t, 0))
        out_desc(t, d).start()

    for dsc in in_descs(0, d0):
        dsc.start()

    def ring(i, _):
        per_d(0, 2 * i, i == 0)
        per_d(1, 2 * i + 1, i == 0)
        return 0

    lax.fori_loop(0, D_PER_W // 2, ring, 0)
    for dsc in in_descs(0, d0):
        dsc.wait()
    out_desc(0, 0).wait()
    out_desc(1, 0).wait()


@jax.jit
def kernel(X, params):
    xt = jnp.transpose(X, (1, 2, 0)).reshape(D, ARITY * B)
    pr = params.reshape(D, NMASK * OUT)

    run = pl.kernel(
        _sc_body,
        out_type=jax.ShapeDtypeStruct((D, B * OUT), jnp.float32),
        mesh=plsc.VectorSubcoreMesh(core_axis_name="c", subcore_axis_name="s"),
        compiler_params=pltpu.CompilerParams(needs_layout_passes=False),
        scratch_types=[
            pltpu.VMEM((2 * _XB,), jnp.float32),
            pltpu.VMEM((2 * _PB,), jnp.float32),
            pltpu.VMEM((2 * _OB,), jnp.float32),
            pltpu.SemaphoreType.DMA,
            pltpu.SemaphoreType.DMA,
            pltpu.SemaphoreType.DMA,
            pltpu.SemaphoreType.DMA,
        ],
    )
    out_t = run(xt, pr)
    return jnp.transpose(out_t.reshape(D, B, OUT), (1, 0, 2))
